# baseline (speedup 1.0000x reference)
import functools

import jax
import jax.numpy as jnp
from jax import lax
from jax.experimental import pallas as pl
from jax.experimental.pallas import tpu as pltpu

B, C, T, H, W_SP = 16, 192, 64, 14, 14
HW = H * W_SP
HID, EMB, K = 384, 32, 8
CBLK = 192

_F32 = jnp.float32
_HI = lax.Precision.HIGHEST


def _hswish(v):
    return v * jnp.clip(v + 3.0, 0.0, 6.0) * jnp.float32(1.0 / 6.0)


def _tree(xs):
    xs = list(xs)
    while len(xs) > 1:
        xs = [xs[i] + xs[i + 1] if i + 1 < len(xs) else xs[i]
              for i in range(0, len(xs), 2)]
    return xs[0]


def _tree_max(xs):
    xs = list(xs)
    while len(xs) > 1:
        xs = [jnp.maximum(xs[i], xs[i + 1]) if i + 1 < len(xs) else xs[i]
              for i in range(0, len(xs), 2)]
    return xs[0]


def _pool_body(x_ref, out_ref):
    xb = x_ref[0]
    slices = [xb[h, w] for h in range(H) for w in range(W_SP)]
    gsums = [_tree(slices[s:s + 64]) for s in range(0, HW, 64)]
    acc = gsums[0]
    for g in gsums[1:]:
        acc = acc + g
    mx = _tree_max(slices)
    out_ref[0] = acc * jnp.float32(1.0 / HW) + mx


def _mxu_emul(a, b):
    kdim = b.shape[0]
    chunks = []
    for c0 in range(0, kdim, 256):
        kk = min(256, kdim - c0)
        gsums = []
        for j in range(16):
            base = j * 8
            leaves = []
            for i in range(8):
                for off in (0, 128):
                    k = base + i + off
                    if k < kk:
                        leaves.append(a[:, c0 + k:c0 + k + 1] * b[c0 + k:c0 + k + 1, :])
            if leaves:
                gsums.append(_tree(leaves))
        ch = gsums[0]
        for g in gsums[1:]:
            ch = ch + g
        chunks.append(ch)
    out = chunks[0]
    for ch in chunks[1:]:
        out = out + ch
    return out


def _logic_math(pt, w1t, b1r, w2t, b2r):
    ptb = pt.astype(jnp.bfloat16).astype(_F32)
    w1tb = w1t.astype(jnp.bfloat16).astype(_F32)
    hdd = _mxu_emul(ptb, w1tb) + b1r
    hdd = _hswish(hdd)
    hddb = hdd.astype(jnp.bfloat16).astype(_F32)
    w2tb = w2t.astype(jnp.bfloat16).astype(_F32)
    embt = _mxu_emul(hddb, w2tb) + b2r
    emb = embt.T
    nrm = jnp.sqrt(jnp.sum(emb * emb, axis=0, keepdims=True))
    ne = emb / jnp.maximum(nrm, 1e-12)

    neigh = jnp.sum(ne[:, 1:] * ne[:, :-1], axis=0, keepdims=True)
    vals = jnp.concatenate([neigh, jnp.full((1, 1), jnp.inf, _F32)], axis=1)

    iota_r = lax.broadcasted_iota(jnp.int32, (1, T), 1)
    breaks = jnp.zeros((1, T), _F32)
    for _ in range(K - 1):
        m = jnp.min(vals, axis=1, keepdims=True)
        cand = jnp.where(vals == m, iota_r, 10**9)
        idx = jnp.min(cand, axis=1, keepdims=True)
        hit = iota_r == idx
        breaks = jnp.where(hit, 1.0, breaks)
        vals = jnp.where(hit, jnp.inf, vals)

    ends = jnp.concatenate([jnp.zeros((1, 1), _F32), breaks[:, : T - 1]], axis=1)
    lower = (
        lax.broadcasted_iota(jnp.int32, (T, T), 0)
        <= lax.broadcasted_iota(jnp.int32, (T, T), 1)
    ).astype(_F32)
    groups_c = lax.dot_general(
        lower, ends, (((0,), (1,)), ((), ())),
        preferred_element_type=_F32, precision=_HI,
    )

    groups_i = groups_c.astype(jnp.int32)
    onehot = (
        jnp.broadcast_to(groups_i, (T, K))
        == lax.broadcasted_iota(jnp.int32, (T, K), 1)
    ).astype(_F32)

    prod = ne[:, :, None] * onehot[None, :, :]
    cs = jnp.sum(prod, axis=1)
    sizes = jnp.sum(onehot, axis=0, keepdims=True)
    cmean = cs / sizes
    cnrm = jnp.sqrt(jnp.sum(cmean * cmean, axis=0, keepdims=True))
    cn = cmean / jnp.maximum(cnrm, 1e-12)

    sims3 = ne[:, :, None] * cn[:, None, :]
    sims = jnp.clip(jnp.sum(sims3, axis=0), -1.0, 1.0)
    val = sims * onehot
    iota_t = lax.broadcasted_iota(jnp.int32, (T, K), 0)
    mx = jnp.max(val, axis=0, keepdims=True)
    cand = jnp.where(val == mx, iota_t, 10**6)
    pos = jnp.min(cand, axis=0, keepdims=True)
    return emb, neigh, val, pos


def _logic_body(pooled_ref, w1t_ref, b1_ref, w2t_ref, b2_ref, pos_ref):
    pt = pooled_ref[0]
    _, _, _, pos = _logic_math(pt, w1t_ref[...], b1_ref[...], w2t_ref[...], b2_ref[...])
    pos_ref[0] = pos


def _gather_body(pos_sref, x_ref, out_ref):
    del pos_sref
    out_ref[...] = x_ref[...]


def _pos_only(x, W1, b1, W2, b2):
    xt = jnp.transpose(x, (0, 3, 4, 2, 1))
    pooled = pl.pallas_call(
        _pool_body,
        grid=(B, C // CBLK),
        in_specs=[
            pl.BlockSpec((1, H, W_SP, T, CBLK), lambda b, c: (b, 0, 0, 0, c)),
        ],
        out_specs=pl.BlockSpec((1, T, CBLK), lambda b, c: (b, 0, c)),
        out_shape=jax.ShapeDtypeStruct((B, T, C), _F32),
    )(xt)

    pos = pl.pallas_call(
        _logic_body,
        grid=(B,),
        in_specs=[
            pl.BlockSpec((1, T, C), lambda b: (b, 0, 0)),
            pl.BlockSpec((C, HID), lambda b: (0, 0)),
            pl.BlockSpec((1, HID), lambda b: (0, 0)),
            pl.BlockSpec((HID, EMB), lambda b: (0, 0)),
            pl.BlockSpec((1, EMB), lambda b: (0, 0)),
        ],
        out_specs=pl.BlockSpec((1, 1, K), lambda b: (b, 0, 0)),
        out_shape=jax.ShapeDtypeStruct((B, 1, K), jnp.int32),
    )(pooled, W1.T, b1[None, :], W2.T, b2[None, :])

    return pos.reshape(B, K)


@jax.jit
def kernel(x, W1, b1, W2, b2):
    pos_flat = _pos_only(x, W1, b1, W2, b2)

    out = pl.pallas_call(
        _gather_body,
        grid_spec=pltpu.PrefetchScalarGridSpec(
            num_scalar_prefetch=1,
            grid=(B, K),
            in_specs=[
                pl.BlockSpec(
                    (1, C, 1, H, W_SP), lambda b, k, pos_s: (b, 0, pos_s[b, k], 0, 0)
                ),
            ],
            out_specs=pl.BlockSpec(
                (1, C, 1, H, W_SP), lambda b, k, pos_s: (b, 0, k, 0, 0)
            ),
        ),
        out_shape=jax.ShapeDtypeStruct((B, C, K, H, W_SP), _F32),
    )(pos_flat, x)
    return out
